# baseline (speedup 1.0000x reference)
import jax
import jax.numpy as jnp
from jax.experimental import pallas as pl
from jax.experimental.pallas import tpu as pltpu

DEPTH = 7
H_BLK = 256
NJ = 512 // H_BLK
NSTEPS = 16 * NJ
IMG_CHUNKS = 8
W_CHUNKS = 4


def _body(mask_hbm, img_hbm, w_hbm, oh_ref, img_out, w_out,
          mask_vmem, msem, img_sems, w_sems):
    i = pl.program_id(0)
    j = pl.program_id(1)
    step = i * NJ + j

    @pl.when(step < IMG_CHUNKS)
    def _():
        c = step
        pltpu.make_async_copy(
            img_hbm.at[pl.ds(c * 2, 2)],
            img_out.at[pl.ds(c * 2, 2)],
            img_sems.at[c],
        ).start()

    @pl.when(step < W_CHUNKS)
    def _():
        c = step
        pltpu.make_async_copy(
            w_hbm.at[pl.ds(c * 4, 4)],
            w_out.at[pl.ds(c * 4, 4)],
            w_sems.at[c],
        ).start()

    slot = jax.lax.rem(step, 2)
    nslot = jax.lax.rem(step + 1, 2)

    @pl.when(step == 0)
    def _():
        pltpu.make_async_copy(
            mask_hbm.at[0, pl.ds(0, H_BLK)], mask_vmem.at[0], msem.at[0]
        ).start()

    @pl.when(step + 1 < NSTEPS)
    def _():
        nstep = step + 1
        ni = jax.lax.div(nstep, NJ)
        njj = jax.lax.rem(nstep, NJ)
        pltpu.make_async_copy(
            mask_hbm.at[ni, pl.ds(njj * H_BLK, H_BLK)],
            mask_vmem.at[nslot],
            msem.at[nslot],
        ).start()

    pltpu.make_async_copy(
        mask_hbm.at[i, pl.ds(j * H_BLK, H_BLK)], mask_vmem.at[slot], msem.at[slot]
    ).wait()

    m = mask_vmem[slot]
    for c in range(DEPTH):
        oh_ref[0, c] = (m == c).astype(jnp.float32)

    @pl.when(step == NSTEPS - 1)
    def _():
        for c in range(IMG_CHUNKS):
            pltpu.make_async_copy(
                img_hbm.at[pl.ds(c * 2, 2)],
                img_out.at[pl.ds(c * 2, 2)],
                img_sems.at[c],
            ).wait()
        for c in range(W_CHUNKS):
            pltpu.make_async_copy(
                w_hbm.at[pl.ds(c * 4, 4)],
                w_out.at[pl.ds(c * 4, 4)],
                w_sems.at[c],
            ).wait()


@jax.jit
def _fused(masks, img_t, weights):
    b, h, w = masks.shape
    masks = pltpu.with_memory_space_constraint(masks, pltpu.MemorySpace.HBM)
    return pl.pallas_call(
        _body,
        grid=(b, NJ),
        in_specs=[
            pl.BlockSpec(memory_space=pltpu.MemorySpace.HBM),
            pl.BlockSpec(memory_space=pltpu.MemorySpace.HBM),
            pl.BlockSpec(memory_space=pltpu.MemorySpace.HBM),
        ],
        out_specs=[
            pl.BlockSpec((1, DEPTH, H_BLK, w), lambda i, j: (i, 0, j, 0)),
            pl.BlockSpec(memory_space=pl.ANY),
            pl.BlockSpec(memory_space=pl.ANY),
        ],
        out_shape=[
            jax.ShapeDtypeStruct((b, DEPTH, h, w), jnp.float32),
            jax.ShapeDtypeStruct(img_t.shape, img_t.dtype),
            jax.ShapeDtypeStruct(weights.shape, weights.dtype),
        ],
        scratch_shapes=[
            pltpu.VMEM((2, H_BLK, w), jnp.int32),
            pltpu.SemaphoreType.DMA((2,)),
            pltpu.SemaphoreType.DMA((IMG_CHUNKS,)),
            pltpu.SemaphoreType.DMA((W_CHUNKS,)),
        ],
    )(masks, img_t, weights)


def kernel(images, masks, weights):
    img_t = jnp.transpose(images, (0, 3, 1, 2))
    oh_planes, img_out, w_out = _fused(masks, img_t, weights)
    return (
        jnp.transpose(img_out, (0, 2, 3, 1)),
        jnp.transpose(oh_planes, (0, 2, 3, 1)),
        w_out,
    )


# speedup vs baseline: 21.9412x; 21.9412x over previous
import jax
import jax.numpy as jnp
from jax.experimental import pallas as pl
from jax.experimental.pallas import tpu as pltpu

DEPTH = 7
H_BLK = 256
NJ = 512 // H_BLK


def _body(mask_ref, oh_ref):
    m = mask_ref[0]
    for c in range(DEPTH):
        oh_ref[0, c] = (m == c).astype(jnp.float32)


@jax.jit
def _one_hot_planes(masks):
    b, h, w = masks.shape
    masks = pltpu.with_memory_space_constraint(masks, pltpu.MemorySpace.HBM)
    return pl.pallas_call(
        _body,
        grid=(b, NJ),
        in_specs=[pl.BlockSpec((1, H_BLK, w), lambda i, j: (i, j, 0))],
        out_specs=pl.BlockSpec((1, DEPTH, H_BLK, w), lambda i, j: (i, 0, j, 0)),
        out_shape=jax.ShapeDtypeStruct((b, DEPTH, h, w), jnp.float32),
        compiler_params=pltpu.CompilerParams(
            dimension_semantics=("parallel", "parallel"),
        ),
    )(masks)


def kernel(images, masks, weights):
    oh_planes = _one_hot_planes(masks)
    return (images, jnp.transpose(oh_planes, (0, 2, 3, 1)), weights)


# speedup vs baseline: 23.4234x; 1.0676x over previous
import jax
import jax.numpy as jnp
from jax.experimental import pallas as pl
from jax.experimental.pallas import tpu as pltpu

DEPTH = 7
H_BLK = 256
NJ = 512 // H_BLK


def _body(mask_ref, img_ref, w_ref, oh_ref, img_out, w_out):
    m = mask_ref[0]
    for c in range(DEPTH):
        oh_ref[0, c] = (m == c).astype(jnp.float32)
    img_out[...] = img_ref[...]
    w_out[...] = w_ref[...]


@jax.jit
def _fused(masks, img_t, weights):
    b, h, w = masks.shape
    masks = pltpu.with_memory_space_constraint(masks, pltpu.MemorySpace.HBM)
    img_t = pltpu.with_memory_space_constraint(img_t, pltpu.MemorySpace.HBM)
    weights = pltpu.with_memory_space_constraint(weights, pltpu.MemorySpace.HBM)
    return pl.pallas_call(
        _body,
        grid=(b, NJ),
        in_specs=[
            pl.BlockSpec((1, H_BLK, w), lambda i, j: (i, j, 0)),
            pl.BlockSpec((1, 3, H_BLK, w), lambda i, j: (i, 0, j, 0)),
            pl.BlockSpec((1, H_BLK, w), lambda i, j: (i, j, 0)),
        ],
        out_specs=[
            pl.BlockSpec((1, DEPTH, H_BLK, w), lambda i, j: (i, 0, j, 0)),
            pl.BlockSpec((1, 3, H_BLK, w), lambda i, j: (i, 0, j, 0)),
            pl.BlockSpec((1, H_BLK, w), lambda i, j: (i, j, 0)),
        ],
        out_shape=[
            jax.ShapeDtypeStruct((b, DEPTH, h, w), jnp.float32),
            jax.ShapeDtypeStruct(img_t.shape, img_t.dtype),
            jax.ShapeDtypeStruct(weights.shape, weights.dtype),
        ],
        compiler_params=pltpu.CompilerParams(
            dimension_semantics=("parallel", "parallel"),
        ),
    )(masks, img_t, weights)


def kernel(images, masks, weights):
    img_t = jnp.transpose(images, (0, 3, 1, 2))
    oh_planes, img_out, w_out = _fused(masks, img_t, weights)
    return (
        jnp.transpose(img_out, (0, 2, 3, 1)),
        jnp.transpose(oh_planes, (0, 2, 3, 1)),
        w_out,
    )
